# baseline (speedup 1.0000x reference)
import functools

import jax
import jax.numpy as jnp
from jax import lax
from jax.experimental import pallas as pl
from jax.experimental.pallas import tpu as pltpu
from jax.experimental.pallas import tpu_sc as plsc

_NUM_WORKERS = 32
_BLK = 128
_NBUF = 4
_PITCH = 129


@functools.cache
def _make_gather(hist: int, vocab: int, dim: int, n_btiles: int):
    n_blocks = hist * n_btiles
    blocks_per_w = n_blocks // _NUM_WORKERS
    assert blocks_per_w * _NUM_WORKERS == n_blocks
    assert blocks_per_w % _NBUF == 0
    c_tiles = dim // 8

    mesh = plsc.VectorSubcoreMesh(core_axis_name="c", subcore_axis_name="s")

    @functools.partial(
        pl.kernel,
        out_type=jax.ShapeDtypeStruct((hist, c_tiles, n_btiles, 1024),
                                      jnp.float32),
        mesh=mesh,
        scratch_types=[
            [pltpu.VMEM((_BLK,), jnp.int32)] * _NBUF,
            [pltpu.VMEM((_BLK, 128), jnp.float32)] * _NBUF,
            [pltpu.VMEM((dim, _PITCH), jnp.float32)] * _NBUF,
            [pltpu.SemaphoreType.DMA] * _NBUF,
            [pltpu.SemaphoreType.DMA] * _NBUF,
            [pltpu.SemaphoreType.DMA] * _NBUF,
        ],
        compiler_params=pltpu.CompilerParams(use_tc_tiling_on_sc=False,
                                             needs_layout_passes=False),
    )
    def gather_kernel(wordsT_hbm, table_hbm, out_hbm,
                      idx_v, rows_v, trans_v, isems, gsems, ssems):
        wid = lax.axis_index("s") * 2 + lax.axis_index("c")
        base = wid * blocks_per_w
        lanes = lax.iota(jnp.int32, 16)

        def idx_src(t):
            blk = base + t
            h = blk // n_btiles
            b_hi = blk % n_btiles
            return wordsT_hbm.at[h, pl.ds(b_hi * _BLK, _BLK)]

        def drain_stores(b):
            pltpu.make_async_copy(table_hbm.at[pl.ds(0, dim)],
                                  rows_v[b].at[pl.ds(0, dim)], ssems[b]).wait()

        for b in range(_NBUF):
            pltpu.sync_copy(idx_src(b), idx_v[b])
            pltpu.async_copy(table_hbm.at[idx_v[b]], rows_v[b], gsems[b])

        @pl.loop(0, blocks_per_w, step=_NBUF)
        def _super(g):
            for b in range(_NBUF):
                t = g + b
                blk = base + t
                h = blk // n_btiles
                b_hi = blk % n_btiles
                pltpu.make_async_copy(table_hbm.at[idx_v[b]], rows_v[b],
                                      gsems[b]).wait()

                @pl.when(g > 0)
                def _free_trans():
                    drain_stores(b)

                @plsc.parallel_loop(0, _BLK, unroll=8)
                def _transpose(bl):
                    bvec = jnp.full((16,), bl, jnp.int32)
                    x0 = rows_v[b][bl, pl.ds(0, 16)]
                    x1 = rows_v[b][bl, pl.ds(16, 16)]
                    plsc.store_scatter(trans_v[b], [lanes, bvec], x0)
                    plsc.store_scatter(trans_v[b], [lanes + 16, bvec], x1)

                for c in range(dim):
                    pltpu.async_copy(
                        trans_v[b].at[c, pl.ds(0, _BLK)],
                        out_hbm.at[h, c // 8, b_hi,
                                   pl.ds((c % 8) * _BLK, _BLK)],
                        ssems[b])

                @pl.when(g + _NBUF < blocks_per_w)
                def _refill():
                    pltpu.async_copy(idx_src(t + _NBUF), idx_v[b], isems[b])
                    pltpu.make_async_copy(idx_src(t + _NBUF), idx_v[b],
                                          isems[b]).wait()
                    pltpu.async_copy(table_hbm.at[idx_v[b]], rows_v[b],
                                     gsems[b])

        for b in range(_NBUF):
            drain_stores(b)

    return gather_kernel


def kernel(words, table):
    batch, hist = words.shape
    vocab, dim = table.shape
    wordsT = words.T.astype(jnp.int32)
    table_pad = jnp.pad(table, ((0, 0), (0, 128 - dim)))
    n_btiles = batch // _BLK
    out5 = _make_gather(hist, vocab, dim, n_btiles)(wordsT, table_pad)
    out = out5.reshape(hist, dim // 8, n_btiles, 8, _BLK)
    out = out.transpose(2, 4, 0, 1, 3).reshape(batch, hist, dim)
    return out


# speedup vs baseline: 1.0409x; 1.0409x over previous
import functools

import jax
import jax.numpy as jnp
from jax import lax
from jax.experimental import pallas as pl
from jax.experimental.pallas import tpu as pltpu
from jax.experimental.pallas import tpu_sc as plsc

_NUM_WORKERS = 32
_BLK = 128
_NBUF = 4
_PITCH = 129


@functools.cache
def _make_gather(hist: int, vocab: int, dim: int, n_btiles: int):
    n_blocks = hist * n_btiles
    blocks_per_w = n_blocks // _NUM_WORKERS
    assert blocks_per_w * _NUM_WORKERS == n_blocks
    assert blocks_per_w % _NBUF == 0
    c_tiles = dim // 8

    mesh = plsc.VectorSubcoreMesh(core_axis_name="c", subcore_axis_name="s")

    @functools.partial(
        pl.kernel,
        out_type=jax.ShapeDtypeStruct((hist, c_tiles, n_btiles, 1024),
                                      jnp.float32),
        mesh=mesh,
        scratch_types=[
            [pltpu.VMEM((_BLK,), jnp.int32)] * _NBUF,
            [pltpu.VMEM((_BLK, dim), jnp.float32)] * _NBUF,
            [pltpu.VMEM((dim, _PITCH), jnp.float32)] * _NBUF,
            [pltpu.SemaphoreType.DMA] * _NBUF,
            [pltpu.SemaphoreType.DMA] * _NBUF,
            [pltpu.SemaphoreType.DMA] * _NBUF,
        ],
        compiler_params=pltpu.CompilerParams(use_tc_tiling_on_sc=False,
                                             needs_layout_passes=False),
    )
    def gather_kernel(wordsT_hbm, table_hbm, out_hbm,
                      idx_v, rows_v, trans_v, isems, gsems, ssems):
        wid = lax.axis_index("s") * 2 + lax.axis_index("c")
        base = wid * blocks_per_w
        lanes = lax.iota(jnp.int32, 16)

        def idx_src(t):
            blk = base + t
            h = blk // n_btiles
            b_hi = blk % n_btiles
            return wordsT_hbm.at[h, pl.ds(b_hi * _BLK, _BLK)]

        def drain_stores(b):
            pltpu.make_async_copy(table_hbm.at[pl.ds(0, _BLK)],
                                  rows_v[b], ssems[b]).wait()

        def quadify(b):
            for grp in range(_BLK // 16):
                sl = pl.ds(grp * 16, 16)
                idx_v[b][sl] = lax.shift_left(idx_v[b][sl], 2)

        for b in range(_NBUF):
            pltpu.sync_copy(idx_src(b), idx_v[b])
            quadify(b)
            pltpu.async_copy(table_hbm.at[idx_v[b]], rows_v[b], gsems[b])

        @pl.loop(0, blocks_per_w, step=_NBUF)
        def _super(g):
            for b in range(_NBUF):
                t = g + b
                blk = base + t
                h = blk // n_btiles
                b_hi = blk % n_btiles
                pltpu.make_async_copy(table_hbm.at[idx_v[b]], rows_v[b],
                                      gsems[b]).wait()

                @pl.when(g > 0)
                def _free_trans():
                    drain_stores(b)

                @plsc.parallel_loop(0, _BLK, unroll=8)
                def _transpose(bl):
                    bvec = jnp.full((16,), bl, jnp.int32)
                    x0 = rows_v[b][bl, pl.ds(0, 16)]
                    x1 = rows_v[b][bl, pl.ds(16, 16)]
                    plsc.store_scatter(trans_v[b], [lanes, bvec], x0)
                    plsc.store_scatter(trans_v[b], [lanes + 16, bvec], x1)

                for c in range(dim):
                    pltpu.async_copy(
                        trans_v[b].at[c, pl.ds(0, _BLK)],
                        out_hbm.at[h, c // 8, b_hi,
                                   pl.ds((c % 8) * _BLK, _BLK)],
                        ssems[b])

                @pl.when(g + _NBUF < blocks_per_w)
                def _refill():
                    pltpu.async_copy(idx_src(t + _NBUF), idx_v[b], isems[b])
                    pltpu.make_async_copy(idx_src(t + _NBUF), idx_v[b],
                                          isems[b]).wait()
                    quadify(b)
                    pltpu.async_copy(table_hbm.at[idx_v[b]], rows_v[b],
                                     gsems[b])

        for b in range(_NBUF):
            drain_stores(b)

    return gather_kernel


def kernel(words, table):
    batch, hist = words.shape
    vocab, dim = table.shape
    wordsT = words.T.astype(jnp.int32)
    table_q = jnp.pad(table, ((0, 0), (0, 128 - dim))).reshape(4 * vocab, dim)
    n_btiles = batch // _BLK
    out5 = _make_gather(hist, vocab, dim, n_btiles)(wordsT, table_q)
    out = out5.reshape(hist, dim // 8, n_btiles, 8, _BLK)
    out = out.transpose(2, 4, 0, 1, 3).reshape(batch, hist, dim)
    return out


# speedup vs baseline: 1.7429x; 1.6744x over previous
import functools

import jax
import jax.numpy as jnp
from jax import lax
from jax.experimental import pallas as pl
from jax.experimental.pallas import tpu as pltpu
from jax.experimental.pallas import tpu_sc as plsc

_NUM_WORKERS = 32
_BLK = 128
_NBUF = 4
_PITCH = 129
_SPITCH = 33
_CNBUF = 3


@functools.cache
def _make_converter(vocab_pad: int, dim: int):
    c_tiles = dim // 8
    n_rhi = vocab_pad // 128
    per_w = -(-n_rhi // _NUM_WORKERS)

    mesh = plsc.VectorSubcoreMesh(core_axis_name="c", subcore_axis_name="s")

    @functools.partial(
        pl.kernel,
        out_type=jax.ShapeDtypeStruct((vocab_pad, dim), jnp.float32),
        mesh=mesh,
        scratch_types=[
            [[pltpu.VMEM((8, 128), jnp.float32)] * c_tiles] * _CNBUF,
            [pltpu.VMEM((128, _SPITCH), jnp.float32)] * _CNBUF,
            [pltpu.VMEM((128, 32), jnp.float32)] * _CNBUF,
            [pltpu.SemaphoreType.DMA] * _CNBUF,
            [pltpu.SemaphoreType.DMA] * _CNBUF,
        ],
        compiler_params=pltpu.CompilerParams(use_tc_tiling_on_sc=False,
                                             needs_layout_passes=False),
    )
    def converter_kernel(tq_hbm, scr_hbm, tile_v, ptr_v, pcp_v, gsems, ssems):
        wid = lax.axis_index("s") * 2 + lax.axis_index("c")
        lanes = lax.iota(jnp.int32, 16)

        def rhi_of(j):
            return wid + j * _NUM_WORKERS

        def load_tiles(j, s):
            rhi = rhi_of(j)

            @pl.when(rhi < n_rhi)
            def _():
                for ci in range(c_tiles):
                    pltpu.async_copy(tq_hbm.at[ci, rhi], tile_v[s][ci],
                                     gsems[s])

        def wait_tiles(s):
            for ci in range(c_tiles):
                pltpu.make_async_copy(tq_hbm.at[0, 0], tile_v[s][ci],
                                      gsems[s]).wait()

        def drain_store(s):
            pltpu.make_async_copy(scr_hbm.at[pl.ds(0, 128)], pcp_v[s],
                                  ssems[s]).wait()

        for s in range(_CNBUF):
            load_tiles(s, s)

        @pl.loop(0, per_w, step=_CNBUF)
        def _outer(g):
            for s in range(_CNBUF):
                j = g + s
                rhi = rhi_of(j)

                @pl.when(rhi < n_rhi)
                def _do():
                    wait_tiles(s)

                    @pl.when(g > 0)
                    def _fr():
                        drain_store(s)

                    for ci in range(c_tiles):
                        @plsc.parallel_loop(0, 8, unroll=2)
                        def _cl(cl):
                            cvec = jnp.full((16,), ci * 8 + cl, jnp.int32)
                            for rg in range(8):
                                x = tile_v[s][ci][cl, pl.ds(rg * 16, 16)]
                                plsc.store_scatter(
                                    ptr_v[s], [lanes + rg * 16, cvec], x)

                    @plsc.parallel_loop(0, 128, unroll=8)
                    def _cp(r):
                        pcp_v[s][r, pl.ds(0, 16)] = ptr_v[s][r, pl.ds(0, 16)]
                        pcp_v[s][r, pl.ds(16, 16)] = ptr_v[s][r, pl.ds(16, 16)]

                    pltpu.async_copy(pcp_v[s],
                                     scr_hbm.at[pl.ds(rhi * 128, 128)],
                                     ssems[s])
                    load_tiles(j + _CNBUF, s)

        for s in range(_CNBUF):
            drain_store(s)

    return converter_kernel


@functools.cache
def _make_gather(hist: int, dim: int, n_btiles: int, vocab_pad: int):
    n_blocks = hist * n_btiles
    blocks_per_w = n_blocks // _NUM_WORKERS
    assert blocks_per_w * _NUM_WORKERS == n_blocks
    assert blocks_per_w % _NBUF == 0
    c_tiles = dim // 8

    mesh = plsc.VectorSubcoreMesh(core_axis_name="c", subcore_axis_name="s")

    @functools.partial(
        pl.kernel,
        out_type=jax.ShapeDtypeStruct((hist, c_tiles, n_btiles, 1024),
                                      jnp.float32),
        mesh=mesh,
        scratch_types=[
            [pltpu.VMEM((_BLK,), jnp.int32)] * _NBUF,
            [pltpu.VMEM((_BLK, dim), jnp.float32)] * _NBUF,
            [pltpu.VMEM((dim, _PITCH), jnp.float32)] * _NBUF,
            pltpu.VMEM((4096,), jnp.int32),
            [pltpu.SemaphoreType.DMA] * _NBUF,
            [pltpu.SemaphoreType.DMA] * _NBUF,
            [pltpu.SemaphoreType.DMA] * _NBUF,
        ],
        compiler_params=pltpu.CompilerParams(use_tc_tiling_on_sc=False,
                                             needs_layout_passes=False),
    )
    def gather_kernel(wordsT_hbm, scr_hbm, out_hbm,
                      idx_v, rows_v, trans_v, drain_v, isems, gsems, ssems):
        wid = lax.axis_index("s") * 2 + lax.axis_index("c")
        base = wid * blocks_per_w
        lanes = lax.iota(jnp.int32, 16)

        def idx_src(t):
            blk = base + t
            h = blk // n_btiles
            b_hi = blk % n_btiles
            return wordsT_hbm.at[h, pl.ds(b_hi * _BLK, _BLK)]

        def drain_stores(b):
            pltpu.make_async_copy(wordsT_hbm.at[0, pl.ds(0, 4096)],
                                  drain_v, ssems[b]).wait()

        for b in range(_NBUF):
            pltpu.sync_copy(idx_src(b), idx_v[b])
            pltpu.async_copy(scr_hbm.at[idx_v[b]], rows_v[b], gsems[b])

        @pl.loop(0, blocks_per_w, step=_NBUF)
        def _super(g):
            for b in range(_NBUF):
                t = g + b
                blk = base + t
                h = blk // n_btiles
                b_hi = blk % n_btiles
                pltpu.make_async_copy(scr_hbm.at[idx_v[b]], rows_v[b],
                                      gsems[b]).wait()

                @pl.when(g > 0)
                def _free_trans():
                    drain_stores(b)

                @plsc.parallel_loop(0, _BLK, unroll=8)
                def _transpose(bl):
                    bvec = jnp.full((16,), bl, jnp.int32)
                    x0 = rows_v[b][bl, pl.ds(0, 16)]
                    x1 = rows_v[b][bl, pl.ds(16, 16)]
                    plsc.store_scatter(trans_v[b], [lanes, bvec], x0)
                    plsc.store_scatter(trans_v[b], [lanes + 16, bvec], x1)

                for c in range(dim):
                    pltpu.async_copy(
                        trans_v[b].at[c, pl.ds(0, _BLK)],
                        out_hbm.at[h, c // 8, b_hi,
                                   pl.ds((c % 8) * _BLK, _BLK)],
                        ssems[b])

                @pl.when(g + _NBUF < blocks_per_w)
                def _refill():
                    pltpu.async_copy(idx_src(t + _NBUF), idx_v[b], isems[b])
                    pltpu.make_async_copy(idx_src(t + _NBUF), idx_v[b],
                                          isems[b]).wait()
                    pltpu.async_copy(scr_hbm.at[idx_v[b]], rows_v[b],
                                     gsems[b])

        for b in range(_NBUF):
            drain_stores(b)

    return gather_kernel


def kernel(words, table):
    batch, hist = words.shape
    vocab, dim = table.shape
    wordsT = words.T.astype(jnp.int32)
    vocab_pad = -(-vocab // 128) * 128
    tpad = jnp.pad(table, ((0, vocab_pad - vocab), (0, 0)))
    tq = tpad.T.reshape(dim // 8, 8, vocab_pad // 128, 128)
    tq = tq.transpose(0, 2, 1, 3)
    scr = _make_converter(vocab_pad, dim)(tq)
    n_btiles = batch // _BLK
    out5 = _make_gather(hist, dim, n_btiles, vocab_pad)(wordsT, scr)
    out = out5.reshape(hist, dim // 8, n_btiles, 8, _BLK)
    out = out.transpose(2, 4, 0, 1, 3).reshape(batch, hist, dim)
    return out
